# baseline (speedup 1.0000x reference)
import functools

import jax
import jax.numpy as jnp
from jax import lax
from jax.experimental import pallas as pl
from jax.experimental.pallas import tpu as pltpu

EPS = 1e-5
BF16 = jnp.bfloat16
F32 = jnp.float32


def _turn_kernel(x_ref, o_ref):
    o_ref[...] = jnp.transpose(x_ref[...].astype(BF16), (1, 0))


def _conv_body(xb, w_ref, maskm_ref, maskp_ref, bpb, cin, s, w, cout):
    xt = jnp.transpose(xb, (0, 2, 1))
    zrow = jnp.zeros((bpb, w, cin), BF16)
    dn = jnp.concatenate([zrow, xt[:, :-w, :]], axis=1)
    up = jnp.concatenate([xt[:, w:, :], zrow], axis=1)
    x3 = jnp.concatenate([dn, xt, up], axis=2)
    p = jnp.dot(x3.reshape(bpb * s, 3 * cin), w_ref[...],
                preferred_element_type=F32)
    pm, p0, pp = p[:, :cout], p[:, cout:2 * cout], p[:, 2 * cout:]
    zs = jnp.zeros((1, cout), F32)
    sm = jnp.concatenate([zs, pm[:-1, :]], axis=0).reshape(bpb, s, cout)
    sp = jnp.concatenate([pp[1:, :], zs], axis=0).reshape(bpb, s, cout)
    acc = p0.reshape(bpb, s, cout)
    acc = acc + sm * maskm_ref[...] + sp * maskp_ref[...]
    return acc


def _make_conv_stats_kernel(bpb, cin, s, w, cout):
    def conv_stats_kernel(x_ref, w_ref, maskm_ref, maskp_ref,
                          conv_ref, stats_ref):
        acc = _conv_body(x_ref[...], w_ref, maskm_ref, maskp_ref,
                         bpb, cin, s, w, cout)
        conv_ref[...] = acc.astype(BF16)
        a2 = acc.reshape(bpb * s, cout)
        sm = jnp.sum(a2, axis=0, keepdims=True)
        sq = jnp.sum(a2 * a2, axis=0, keepdims=True)
        pad = jnp.zeros((stats_ref.shape[0] - 2, stats_ref.shape[1]), F32)
        stats_ref[...] = jnp.concatenate([sm, sq, pad], axis=0)
    return conv_stats_kernel


def _bn_relu_kernel(c_ref, scale_ref, shift_ref, o_ref):
    y = c_ref[...].astype(F32) * scale_ref[...] + shift_ref[...]
    o_ref[...] = jnp.maximum(y, 0.0)


def _pick_bpb(n):
    for cand in (64, 32, 16, 8, 4, 2):
        if n % cand == 0:
            return cand
    return n


@jax.jit
def _chess_conv_block(x_nchw, w_oihw, gamma, beta):
    n, cin, h, w = x_nchw.shape
    cout = w_oihw.shape[0]
    s = h * w

    x_native = jnp.transpose(x_nchw, (1, 2, 3, 0)).reshape(cin * s, n)

    w_all = (jnp.transpose(w_oihw, (2, 1, 3, 0))
             .reshape(3 * cin, 3 * cout).astype(BF16))

    wpos = jnp.arange(s) % w
    maskm = jnp.broadcast_to((wpos != 0).astype(F32)[:, None], (s, cout))
    maskp = jnp.broadcast_to((wpos != w - 1).astype(F32)[:, None], (s, cout))

    bpb = _pick_bpb(n)
    nb = n // bpb

    cparams = pltpu.CompilerParams(
        dimension_semantics=("parallel",),
        vmem_limit_bytes=64 * 1024 * 1024)

    cparams_arb = pltpu.CompilerParams(
        dimension_semantics=("arbitrary",),
        vmem_limit_bytes=64 * 1024 * 1024)

    tb = 256
    while n % tb:
        tb //= 2
    x3 = pl.pallas_call(
        _turn_kernel,
        grid=(n // tb,),
        in_specs=(pl.BlockSpec((cin * s, tb), lambda i: (0, i)),),
        out_specs=pl.BlockSpec((tb, cin * s), lambda i: (i, 0)),
        out_shape=jax.ShapeDtypeStruct((n, cin * s), BF16),
        compiler_params=cparams_arb,
        cost_estimate=pl.CostEstimate(
            flops=0, transcendentals=0,
            bytes_accessed=4 * x_native.size + 2 * x_native.size),
    )(x_native).reshape(n, cin, s)

    conv_flops = 2 * n * s * (3 * cin) * cout * 3
    common_in_specs = (
        pl.BlockSpec((bpb, cin, s), lambda i: (i, 0, 0)),
        pl.BlockSpec((3 * cin, 3 * cout), lambda i: (0, 0)),
        pl.BlockSpec((s, cout), lambda i: (0, 0)),
        pl.BlockSpec((s, cout), lambda i: (0, 0)),
    )

    conv2d, stats = pl.pallas_call(
        _make_conv_stats_kernel(bpb, cin, s, w, cout),
        grid=(nb,),
        in_specs=common_in_specs,
        out_specs=(
            pl.BlockSpec((bpb, s, cout), lambda i: (i, 0, 0)),
            pl.BlockSpec((8, cout), lambda i: (i, 0)),
        ),
        out_shape=(
            jax.ShapeDtypeStruct((n, s, cout), BF16),
            jax.ShapeDtypeStruct((nb * 8, cout), F32),
        ),
        compiler_params=cparams,
        cost_estimate=pl.CostEstimate(
            flops=conv_flops, transcendentals=0,
            bytes_accessed=4 * x3.size + 2 * n * s * cout + 4 * nb * 8 * cout),
    )(x3, w_all, maskm, maskp)

    m_total = n * s
    st = stats.reshape(nb, 8, cout)
    mean = jnp.sum(st[:, 0, :], axis=0) / m_total
    var = jnp.maximum(jnp.sum(st[:, 1, :], axis=0) / m_total - mean * mean, 0.0)
    inv_std = lax.rsqrt(var + EPS)
    scale = gamma.astype(F32) * inv_std
    shift = beta.astype(F32) - mean * scale
    scale3 = scale.reshape(1, 1, cout)
    shift3 = shift.reshape(1, 1, cout)

    out3 = pl.pallas_call(
        _bn_relu_kernel,
        grid=(nb,),
        in_specs=(
            pl.BlockSpec((bpb, s, cout), lambda i: (i, 0, 0)),
            pl.BlockSpec((1, 1, cout), lambda i: (0, 0, 0)),
            pl.BlockSpec((1, 1, cout), lambda i: (0, 0, 0)),
        ),
        out_specs=pl.BlockSpec((bpb, s, cout), lambda i: (i, 0, 0)),
        out_shape=jax.ShapeDtypeStruct((n, s, cout), F32),
        compiler_params=cparams,
        cost_estimate=pl.CostEstimate(
            flops=3 * n * s * cout, transcendentals=0,
            bytes_accessed=6 * n * cout * s),
    )(conv2d, scale3, shift3)

    return jnp.transpose(out3.reshape(n, h, w, cout), (0, 3, 1, 2))


def kernel(x_nchw, w_oihw, b, gamma, beta):
    del b
    return _chess_conv_block(x_nchw, w_oihw, gamma, beta)


# speedup vs baseline: 1.3293x; 1.3293x over previous
import functools

import jax
import jax.numpy as jnp
from jax import lax
from jax.experimental import pallas as pl
from jax.experimental.pallas import tpu as pltpu

EPS = 1e-5
BF16 = jnp.bfloat16
F32 = jnp.float32


def _make_turn_kernel(cin, s):
    def turn_kernel(x_ref, o_ref):
        t = jnp.transpose(x_ref[...].astype(BF16), (1, 0))
        o_ref[...] = t.reshape(t.shape[0], cin, s)
    return turn_kernel


def _conv_body(xb, w_ref, maskm_ref, maskp_ref, bpb, cin, s, w, cout):
    xt = jnp.transpose(xb, (0, 2, 1))
    zrow = jnp.zeros((bpb, w, cin), BF16)
    dn = jnp.concatenate([zrow, xt[:, :-w, :]], axis=1)
    up = jnp.concatenate([xt[:, w:, :], zrow], axis=1)
    x3 = jnp.concatenate([dn, xt, up], axis=2)
    p = jnp.dot(x3.reshape(bpb * s, 3 * cin), w_ref[...],
                preferred_element_type=F32)
    pm, p0, pp = p[:, :cout], p[:, cout:2 * cout], p[:, 2 * cout:]
    zs = jnp.zeros((1, cout), F32)
    sm = jnp.concatenate([zs, pm[:-1, :]], axis=0).reshape(bpb, s, cout)
    sp = jnp.concatenate([pp[1:, :], zs], axis=0).reshape(bpb, s, cout)
    acc = p0.reshape(bpb, s, cout)
    acc = acc + sm * maskm_ref[...] + sp * maskp_ref[...]
    return acc


def _make_conv_stats_kernel(bpb, cin, s, w, cout):
    def conv_stats_kernel(x_ref, w_ref, maskm_ref, maskp_ref,
                          conv_ref, stats_ref):
        acc = _conv_body(x_ref[...], w_ref, maskm_ref, maskp_ref,
                         bpb, cin, s, w, cout)
        conv_ref[...] = acc.astype(BF16)
        a2 = acc.reshape(bpb * s, cout)
        sm = jnp.sum(a2, axis=0, keepdims=True)
        sq = jnp.sum(a2 * a2, axis=0, keepdims=True)
        pad = jnp.zeros((stats_ref.shape[0] - 2, stats_ref.shape[1]), F32)
        stats_ref[...] = jnp.concatenate([sm, sq, pad], axis=0)
    return conv_stats_kernel


def _bn_relu_kernel(c_ref, scale_ref, shift_ref, o_ref):
    y = c_ref[...].astype(F32) * scale_ref[...] + shift_ref[...]
    o_ref[...] = jnp.maximum(y, 0.0)


def _pick_bpb(n):
    for cand in (64, 32, 16, 8, 4, 2):
        if n % cand == 0:
            return cand
    return n


@jax.jit
def _chess_conv_block(x_nchw, w_oihw, gamma, beta):
    n, cin, h, w = x_nchw.shape
    cout = w_oihw.shape[0]
    s = h * w

    x_native = jnp.transpose(x_nchw, (1, 2, 3, 0)).reshape(cin * s, n)

    w_all = (jnp.transpose(w_oihw, (2, 1, 3, 0))
             .reshape(3 * cin, 3 * cout).astype(BF16))

    wpos = jnp.arange(s) % w
    maskm = jnp.broadcast_to((wpos != 0).astype(F32)[:, None], (s, cout))
    maskp = jnp.broadcast_to((wpos != w - 1).astype(F32)[:, None], (s, cout))

    bpb = _pick_bpb(n)
    nb = n // bpb

    cparams = pltpu.CompilerParams(
        dimension_semantics=("parallel",),
        vmem_limit_bytes=64 * 1024 * 1024)

    cparams_arb = pltpu.CompilerParams(
        dimension_semantics=("arbitrary",),
        vmem_limit_bytes=64 * 1024 * 1024)

    tb = 256
    while n % tb:
        tb //= 2
    x3 = pl.pallas_call(
        _make_turn_kernel(cin, s),
        grid=(n // tb,),
        in_specs=(pl.BlockSpec((cin * s, tb), lambda i: (0, i)),),
        out_specs=pl.BlockSpec((tb, cin, s), lambda i: (i, 0, 0)),
        out_shape=jax.ShapeDtypeStruct((n, cin, s), BF16),
        compiler_params=cparams_arb,
        cost_estimate=pl.CostEstimate(
            flops=0, transcendentals=0,
            bytes_accessed=4 * x_native.size + 2 * x_native.size),
    )(x_native)

    conv_flops = 2 * n * s * (3 * cin) * cout * 3
    common_in_specs = (
        pl.BlockSpec((bpb, cin, s), lambda i: (i, 0, 0)),
        pl.BlockSpec((3 * cin, 3 * cout), lambda i: (0, 0)),
        pl.BlockSpec((s, cout), lambda i: (0, 0)),
        pl.BlockSpec((s, cout), lambda i: (0, 0)),
    )

    conv2d, stats = pl.pallas_call(
        _make_conv_stats_kernel(bpb, cin, s, w, cout),
        grid=(nb,),
        in_specs=common_in_specs,
        out_specs=(
            pl.BlockSpec((bpb, s, cout), lambda i: (i, 0, 0)),
            pl.BlockSpec((8, cout), lambda i: (i, 0)),
        ),
        out_shape=(
            jax.ShapeDtypeStruct((n, s, cout), BF16),
            jax.ShapeDtypeStruct((nb * 8, cout), F32),
        ),
        compiler_params=cparams,
        cost_estimate=pl.CostEstimate(
            flops=conv_flops, transcendentals=0,
            bytes_accessed=4 * x3.size + 2 * n * s * cout + 4 * nb * 8 * cout),
    )(x3, w_all, maskm, maskp)

    m_total = n * s
    st = stats.reshape(nb, 8, cout)
    mean = jnp.sum(st[:, 0, :], axis=0) / m_total
    var = jnp.maximum(jnp.sum(st[:, 1, :], axis=0) / m_total - mean * mean, 0.0)
    inv_std = lax.rsqrt(var + EPS)
    scale = gamma.astype(F32) * inv_std
    shift = beta.astype(F32) - mean * scale
    scale3 = scale.reshape(1, 1, cout)
    shift3 = shift.reshape(1, 1, cout)

    out3 = pl.pallas_call(
        _bn_relu_kernel,
        grid=(nb,),
        in_specs=(
            pl.BlockSpec((bpb, s, cout), lambda i: (i, 0, 0)),
            pl.BlockSpec((1, 1, cout), lambda i: (0, 0, 0)),
            pl.BlockSpec((1, 1, cout), lambda i: (0, 0, 0)),
        ),
        out_specs=pl.BlockSpec((bpb, s, cout), lambda i: (i, 0, 0)),
        out_shape=jax.ShapeDtypeStruct((n, s, cout), F32),
        compiler_params=cparams,
        cost_estimate=pl.CostEstimate(
            flops=3 * n * s * cout, transcendentals=0,
            bytes_accessed=6 * n * cout * s),
    )(conv2d, scale3, shift3)

    return jnp.transpose(out3.reshape(n, h, w, cout), (0, 3, 1, 2))


def kernel(x_nchw, w_oihw, b, gamma, beta):
    del b
    return _chess_conv_block(x_nchw, w_oihw, gamma, beta)


# speedup vs baseline: 1.9228x; 1.4465x over previous
import functools

import jax
import jax.numpy as jnp
from jax import lax
from jax.experimental import pallas as pl
from jax.experimental.pallas import tpu as pltpu

EPS = 1e-5
BF16 = jnp.bfloat16
F32 = jnp.float32


def _make_turn_kernel(cin, s):
    def turn_kernel(x_ref, o_ref):
        t = jnp.transpose(x_ref[...].astype(BF16), (1, 0))
        o_ref[...] = jnp.transpose(t.reshape(t.shape[0], cin, s), (0, 2, 1))
    return turn_kernel


def _conv_body(xb, w_ref, maskm_ref, maskp_ref, bpb, cin, s, w, cout):
    xt = xb
    zrow = jnp.zeros((bpb, w, cin), BF16)
    dn = jnp.concatenate([zrow, xt[:, :-w, :]], axis=1)
    up = jnp.concatenate([xt[:, w:, :], zrow], axis=1)
    x3 = jnp.concatenate([dn, xt, up], axis=2)
    p = jnp.dot(x3.reshape(bpb * s, 3 * cin), w_ref[...],
                preferred_element_type=F32)
    pm, p0, pp = p[:, :cout], p[:, cout:2 * cout], p[:, 2 * cout:]
    zs = jnp.zeros((1, cout), F32)
    sm = jnp.concatenate([zs, pm[:-1, :]], axis=0).reshape(bpb, s, cout)
    sp = jnp.concatenate([pp[1:, :], zs], axis=0).reshape(bpb, s, cout)
    acc = p0.reshape(bpb, s, cout)
    acc = acc + sm * maskm_ref[...] + sp * maskp_ref[...]
    return acc


def _make_conv_stats_kernel(bpb, cin, s, w, cout):
    def conv_stats_kernel(x_ref, w_ref, maskm_ref, maskp_ref,
                          conv_ref, stats_ref):
        acc = _conv_body(x_ref[...], w_ref, maskm_ref, maskp_ref,
                         bpb, cin, s, w, cout)
        conv_ref[...] = acc.astype(BF16)
        a2 = acc.reshape(bpb * s, cout)
        sm = jnp.sum(a2, axis=0, keepdims=True)
        sq = jnp.sum(a2 * a2, axis=0, keepdims=True)
        pad = jnp.zeros((stats_ref.shape[0] - 2, stats_ref.shape[1]), F32)
        stats_ref[...] = jnp.concatenate([sm, sq, pad], axis=0)
    return conv_stats_kernel


def _bn_relu_kernel(c_ref, scale_ref, shift_ref, o_ref):
    y = c_ref[...].astype(F32) * scale_ref[...] + shift_ref[...]
    o_ref[...] = jnp.maximum(y, 0.0)


def _pick_bpb(n):
    for cand in (64, 32, 16, 8, 4, 2):
        if n % cand == 0:
            return cand
    return n


@jax.jit
def _chess_conv_block(x_nchw, w_oihw, gamma, beta):
    n, cin, h, w = x_nchw.shape
    cout = w_oihw.shape[0]
    s = h * w

    x_native = jnp.transpose(x_nchw, (1, 2, 3, 0)).reshape(cin * s, n)

    w_all = (jnp.transpose(w_oihw, (2, 1, 3, 0))
             .reshape(3 * cin, 3 * cout).astype(BF16))

    wpos = jnp.arange(s) % w
    maskm = jnp.broadcast_to((wpos != 0).astype(F32)[:, None], (s, cout))
    maskp = jnp.broadcast_to((wpos != w - 1).astype(F32)[:, None], (s, cout))

    bpb = _pick_bpb(n)
    nb = n // bpb

    cparams = pltpu.CompilerParams(
        dimension_semantics=("parallel",),
        vmem_limit_bytes=64 * 1024 * 1024)

    cparams_arb = pltpu.CompilerParams(
        dimension_semantics=("arbitrary",),
        vmem_limit_bytes=64 * 1024 * 1024)

    tb = 256
    while n % tb:
        tb //= 2
    x3 = pl.pallas_call(
        _make_turn_kernel(cin, s),
        grid=(n // tb,),
        in_specs=(pl.BlockSpec((cin * s, tb), lambda i: (0, i)),),
        out_specs=pl.BlockSpec((tb, s, cin), lambda i: (i, 0, 0)),
        out_shape=jax.ShapeDtypeStruct((n, s, cin), BF16),
        compiler_params=cparams_arb,
        cost_estimate=pl.CostEstimate(
            flops=0, transcendentals=0,
            bytes_accessed=4 * x_native.size + 2 * x_native.size),
    )(x_native)

    conv_flops = 2 * n * s * (3 * cin) * cout * 3
    common_in_specs = (
        pl.BlockSpec((bpb, s, cin), lambda i: (i, 0, 0)),
        pl.BlockSpec((3 * cin, 3 * cout), lambda i: (0, 0)),
        pl.BlockSpec((s, cout), lambda i: (0, 0)),
        pl.BlockSpec((s, cout), lambda i: (0, 0)),
    )

    conv2d, stats = pl.pallas_call(
        _make_conv_stats_kernel(bpb, cin, s, w, cout),
        grid=(nb,),
        in_specs=common_in_specs,
        out_specs=(
            pl.BlockSpec((bpb, s, cout), lambda i: (i, 0, 0)),
            pl.BlockSpec((8, cout), lambda i: (i, 0)),
        ),
        out_shape=(
            jax.ShapeDtypeStruct((n, s, cout), BF16),
            jax.ShapeDtypeStruct((nb * 8, cout), F32),
        ),
        compiler_params=cparams,
        cost_estimate=pl.CostEstimate(
            flops=conv_flops, transcendentals=0,
            bytes_accessed=4 * x3.size + 2 * n * s * cout + 4 * nb * 8 * cout),
    )(x3, w_all, maskm, maskp)

    m_total = n * s
    st = stats.reshape(nb, 8, cout)
    mean = jnp.sum(st[:, 0, :], axis=0) / m_total
    var = jnp.maximum(jnp.sum(st[:, 1, :], axis=0) / m_total - mean * mean, 0.0)
    inv_std = lax.rsqrt(var + EPS)
    scale = gamma.astype(F32) * inv_std
    shift = beta.astype(F32) - mean * scale
    scale3 = scale.reshape(1, 1, cout)
    shift3 = shift.reshape(1, 1, cout)

    out3 = pl.pallas_call(
        _bn_relu_kernel,
        grid=(nb,),
        in_specs=(
            pl.BlockSpec((bpb, s, cout), lambda i: (i, 0, 0)),
            pl.BlockSpec((1, 1, cout), lambda i: (0, 0, 0)),
            pl.BlockSpec((1, 1, cout), lambda i: (0, 0, 0)),
        ),
        out_specs=pl.BlockSpec((bpb, s, cout), lambda i: (i, 0, 0)),
        out_shape=jax.ShapeDtypeStruct((n, s, cout), F32),
        compiler_params=cparams,
        cost_estimate=pl.CostEstimate(
            flops=3 * n * s * cout, transcendentals=0,
            bytes_accessed=6 * n * cout * s),
    )(conv2d, scale3, shift3)

    return jnp.transpose(out3.reshape(n, h, w, cout), (0, 3, 1, 2))


def kernel(x_nchw, w_oihw, b, gamma, beta):
    del b
    return _chess_conv_block(x_nchw, w_oihw, gamma, beta)


# speedup vs baseline: 2.0835x; 1.0836x over previous
import functools

import jax
import jax.numpy as jnp
from jax import lax
from jax.experimental import pallas as pl
from jax.experimental.pallas import tpu as pltpu

EPS = 1e-5
BF16 = jnp.bfloat16
F32 = jnp.float32


def _make_turn_kernel(cin, s):
    def turn_kernel(x_ref, o_ref):
        t = jnp.transpose(x_ref[...].astype(BF16), (1, 0))
        o_ref[...] = jnp.transpose(t.reshape(t.shape[0], cin, s), (0, 2, 1))
    return turn_kernel


def _conv_body(xb, w_ref, maskm_ref, maskp_ref, bpb, cin, s, w, cout):
    xt = xb
    zrow = jnp.zeros((bpb, w, cin), BF16)
    dn = jnp.concatenate([zrow, xt[:, :-w, :]], axis=1)
    up = jnp.concatenate([xt[:, w:, :], zrow], axis=1)
    x3 = jnp.concatenate([dn, xt, up], axis=2)
    p = jnp.dot(x3.reshape(bpb * s, 3 * cin), w_ref[...],
                preferred_element_type=F32)
    pm, p0, pp = p[:, :cout], p[:, cout:2 * cout], p[:, 2 * cout:]
    zs = jnp.zeros((1, cout), F32)
    sm = jnp.concatenate([zs, pm[:-1, :]], axis=0).reshape(bpb, s, cout)
    sp = jnp.concatenate([pp[1:, :], zs], axis=0).reshape(bpb, s, cout)
    acc = p0.reshape(bpb, s, cout)
    acc = acc + sm * maskm_ref[...] + sp * maskp_ref[...]
    return acc


def _make_conv_stats_kernel(bpb, cin, s, w, cout):
    def conv_stats_kernel(x_ref, w_ref, maskm_ref, maskp_ref,
                          conv_ref, stats_ref):
        acc = _conv_body(x_ref[...], w_ref, maskm_ref, maskp_ref,
                         bpb, cin, s, w, cout)
        conv_ref[...] = acc.astype(BF16)
        a2 = acc.reshape(bpb * s, cout)
        sm = jnp.sum(a2, axis=0, keepdims=True)
        sq = jnp.sum(a2 * a2, axis=0, keepdims=True)
        pad = jnp.zeros((stats_ref.shape[0] - 2, stats_ref.shape[1]), F32)
        stats_ref[...] = jnp.concatenate([sm, sq, pad], axis=0)
    return conv_stats_kernel


def _bn_relu_kernel(c_ref, scale_ref, shift_ref, o_ref):
    y = c_ref[...].astype(F32) * scale_ref[...] + shift_ref[...]
    o_ref[...] = jnp.maximum(y, 0.0)


def _pick_bpb(n):
    for cand in (64, 32, 16, 8, 4, 2):
        if n % cand == 0:
            return cand
    return n


@jax.jit
def _chess_conv_block(x_nchw, w_oihw, gamma, beta):
    n, cin, h, w = x_nchw.shape
    cout = w_oihw.shape[0]
    s = h * w

    x_native = jnp.transpose(x_nchw, (1, 2, 3, 0)).reshape(cin * s, n)

    w_all = (jnp.transpose(w_oihw, (2, 1, 3, 0))
             .reshape(3 * cin, 3 * cout).astype(BF16))

    wpos = jnp.arange(s) % w
    maskm = jnp.broadcast_to((wpos != 0).astype(F32)[:, None], (s, cout))
    maskp = jnp.broadcast_to((wpos != w - 1).astype(F32)[:, None], (s, cout))

    bpb = _pick_bpb(n)
    nb = n // bpb

    cparams = pltpu.CompilerParams(
        dimension_semantics=("parallel",),
        vmem_limit_bytes=64 * 1024 * 1024)

    cparams_arb = pltpu.CompilerParams(
        dimension_semantics=("arbitrary",),
        vmem_limit_bytes=64 * 1024 * 1024)

    tb = 256
    while n % tb:
        tb //= 2
    x3 = pl.pallas_call(
        _make_turn_kernel(cin, s),
        grid=(n // tb,),
        in_specs=(pl.BlockSpec((cin * s, tb), lambda i: (0, i)),),
        out_specs=pl.BlockSpec((tb, s, cin), lambda i: (i, 0, 0)),
        out_shape=jax.ShapeDtypeStruct((n, s, cin), BF16),
        compiler_params=cparams_arb,
        cost_estimate=pl.CostEstimate(
            flops=0, transcendentals=0,
            bytes_accessed=4 * x_native.size + 2 * x_native.size),
    )(x_native)

    conv_flops = 2 * n * s * (3 * cin) * cout * 3
    common_in_specs = (
        pl.BlockSpec((bpb, s, cin), lambda i: (i, 0, 0)),
        pl.BlockSpec((3 * cin, 3 * cout), lambda i: (0, 0)),
        pl.BlockSpec((s, cout), lambda i: (0, 0)),
        pl.BlockSpec((s, cout), lambda i: (0, 0)),
    )

    conv2d, stats = pl.pallas_call(
        _make_conv_stats_kernel(bpb, cin, s, w, cout),
        grid=(nb,),
        in_specs=common_in_specs,
        out_specs=(
            pl.BlockSpec((bpb, s, cout), lambda i: (i, 0, 0)),
            pl.BlockSpec((8, cout), lambda i: (i, 0)),
        ),
        out_shape=(
            jax.ShapeDtypeStruct((n, s, cout), BF16),
            jax.ShapeDtypeStruct((nb * 8, cout), F32),
        ),
        compiler_params=cparams,
        cost_estimate=pl.CostEstimate(
            flops=conv_flops, transcendentals=0,
            bytes_accessed=4 * x3.size + 2 * n * s * cout + 4 * nb * 8 * cout),
    )(x3, w_all, maskm, maskp)

    m_total = n * s
    st = stats.reshape(nb, 8, cout)
    mean = jnp.sum(st[:, 0, :], axis=0) / m_total
    var = jnp.maximum(jnp.sum(st[:, 1, :], axis=0) / m_total - mean * mean, 0.0)
    inv_std = lax.rsqrt(var + EPS)
    scale = gamma.astype(F32) * inv_std
    shift = beta.astype(F32) - mean * scale
    scale3 = scale.reshape(1, 1, cout)
    shift3 = shift.reshape(1, 1, cout)

    bpb2 = 256
    while n % bpb2:
        bpb2 //= 2
    nb2 = n // bpb2
    out3 = pl.pallas_call(
        _bn_relu_kernel,
        grid=(nb2,),
        in_specs=(
            pl.BlockSpec((bpb2, s, cout), lambda i: (i, 0, 0)),
            pl.BlockSpec((1, 1, cout), lambda i: (0, 0, 0)),
            pl.BlockSpec((1, 1, cout), lambda i: (0, 0, 0)),
        ),
        out_specs=pl.BlockSpec((bpb2, s, cout), lambda i: (i, 0, 0)),
        out_shape=jax.ShapeDtypeStruct((n, s, cout), F32),
        compiler_params=cparams,
        cost_estimate=pl.CostEstimate(
            flops=3 * n * s * cout, transcendentals=0,
            bytes_accessed=6 * n * cout * s),
    )(conv2d, scale3, shift3)

    return jnp.transpose(out3.reshape(n, h, w, cout), (0, 3, 1, 2))


def kernel(x_nchw, w_oihw, b, gamma, beta):
    del b
    return _chess_conv_block(x_nchw, w_oihw, gamma, beta)


# speedup vs baseline: 2.2422x; 1.0762x over previous
import functools

import jax
import jax.numpy as jnp
from jax import lax
from jax.experimental import pallas as pl
from jax.experimental.pallas import tpu as pltpu

EPS = 1e-5
BF16 = jnp.bfloat16
F32 = jnp.float32


def _make_turn_kernel(cin, s):
    def turn_kernel(x_ref, o_ref):
        t = jnp.transpose(x_ref[...].astype(BF16), (1, 0))
        o_ref[...] = jnp.transpose(t.reshape(t.shape[0], cin, s), (0, 2, 1))
    return turn_kernel


def _conv_body(xb, w_ref, maskm_ref, maskp_ref, bpb, cin, s, w, cout):
    xt = xb
    zrow = jnp.zeros((bpb, w, cin), BF16)
    dn = jnp.concatenate([zrow, xt[:, :-w, :]], axis=1)
    up = jnp.concatenate([xt[:, w:, :], zrow], axis=1)
    x3 = jnp.concatenate([dn, xt, up], axis=2)
    p = jnp.dot(x3.reshape(bpb * s, 3 * cin), w_ref[...],
                preferred_element_type=F32)
    pm, p0, pp = p[:, :cout], p[:, cout:2 * cout], p[:, 2 * cout:]
    zs = jnp.zeros((1, cout), F32)
    sm = jnp.concatenate([zs, pm[:-1, :]], axis=0).reshape(bpb, s, cout)
    sp = jnp.concatenate([pp[1:, :], zs], axis=0).reshape(bpb, s, cout)
    acc = p0.reshape(bpb, s, cout)
    acc = acc + sm * maskm_ref[...] + sp * maskp_ref[...]
    return acc


def _make_conv_stats_kernel(bpb, cin, s, w, cout):
    def conv_stats_kernel(x_ref, w_ref, maskm_ref, maskp_ref,
                          conv_ref, stats_ref):
        acc = _conv_body(x_ref[...], w_ref, maskm_ref, maskp_ref,
                         bpb, cin, s, w, cout)
        conv_ref[...] = acc.astype(BF16)
        a2 = acc.reshape(bpb * s, cout)
        sm = jnp.sum(a2, axis=0, keepdims=True)
        sq = jnp.sum(a2 * a2, axis=0, keepdims=True)
        pad = jnp.zeros((stats_ref.shape[0] - 2, stats_ref.shape[1]), F32)
        stats_ref[...] = jnp.concatenate([sm, sq, pad], axis=0)
    return conv_stats_kernel


def _bn_relu_kernel(c_ref, scale_ref, shift_ref, o_ref):
    y = c_ref[...].astype(F32) * scale_ref[...] + shift_ref[...]
    o_ref[...] = jnp.maximum(y, 0.0)


def _pick_bpb(n):
    for cand in (128, 64, 32, 16, 8, 4, 2):
        if n % cand == 0:
            return cand
    return n


@jax.jit
def _chess_conv_block(x_nchw, w_oihw, gamma, beta):
    n, cin, h, w = x_nchw.shape
    cout = w_oihw.shape[0]
    s = h * w

    x_native = jnp.transpose(x_nchw, (1, 2, 3, 0)).reshape(cin * s, n)

    w_all = (jnp.transpose(w_oihw, (2, 1, 3, 0))
             .reshape(3 * cin, 3 * cout).astype(BF16))

    wpos = jnp.arange(s) % w
    maskm = jnp.broadcast_to((wpos != 0).astype(F32)[:, None], (s, cout))
    maskp = jnp.broadcast_to((wpos != w - 1).astype(F32)[:, None], (s, cout))

    bpb = _pick_bpb(n)
    nb = n // bpb

    cparams = pltpu.CompilerParams(
        dimension_semantics=("parallel",),
        vmem_limit_bytes=64 * 1024 * 1024)

    cparams_arb = pltpu.CompilerParams(
        dimension_semantics=("arbitrary",),
        vmem_limit_bytes=64 * 1024 * 1024)

    tb = 256
    while n % tb:
        tb //= 2
    x3 = pl.pallas_call(
        _make_turn_kernel(cin, s),
        grid=(n // tb,),
        in_specs=(pl.BlockSpec((cin * s, tb), lambda i: (0, i)),),
        out_specs=pl.BlockSpec((tb, s, cin), lambda i: (i, 0, 0)),
        out_shape=jax.ShapeDtypeStruct((n, s, cin), BF16),
        compiler_params=cparams_arb,
        cost_estimate=pl.CostEstimate(
            flops=0, transcendentals=0,
            bytes_accessed=4 * x_native.size + 2 * x_native.size),
    )(x_native)

    conv_flops = 2 * n * s * (3 * cin) * cout * 3
    common_in_specs = (
        pl.BlockSpec((bpb, s, cin), lambda i: (i, 0, 0)),
        pl.BlockSpec((3 * cin, 3 * cout), lambda i: (0, 0)),
        pl.BlockSpec((s, cout), lambda i: (0, 0)),
        pl.BlockSpec((s, cout), lambda i: (0, 0)),
    )

    conv2d, stats = pl.pallas_call(
        _make_conv_stats_kernel(bpb, cin, s, w, cout),
        grid=(nb,),
        in_specs=common_in_specs,
        out_specs=(
            pl.BlockSpec((bpb, s, cout), lambda i: (i, 0, 0)),
            pl.BlockSpec((8, cout), lambda i: (i, 0)),
        ),
        out_shape=(
            jax.ShapeDtypeStruct((n, s, cout), BF16),
            jax.ShapeDtypeStruct((nb * 8, cout), F32),
        ),
        compiler_params=cparams,
        cost_estimate=pl.CostEstimate(
            flops=conv_flops, transcendentals=0,
            bytes_accessed=4 * x3.size + 2 * n * s * cout + 4 * nb * 8 * cout),
    )(x3, w_all, maskm, maskp)

    m_total = n * s
    st = stats.reshape(nb, 8, cout)
    mean = jnp.sum(st[:, 0, :], axis=0) / m_total
    var = jnp.maximum(jnp.sum(st[:, 1, :], axis=0) / m_total - mean * mean, 0.0)
    inv_std = lax.rsqrt(var + EPS)
    scale = gamma.astype(F32) * inv_std
    shift = beta.astype(F32) - mean * scale
    scale3 = scale.reshape(1, 1, cout)
    shift3 = shift.reshape(1, 1, cout)

    bpb2 = 256
    while n % bpb2:
        bpb2 //= 2
    nb2 = n // bpb2
    out3 = pl.pallas_call(
        _bn_relu_kernel,
        grid=(nb2,),
        in_specs=(
            pl.BlockSpec((bpb2, s, cout), lambda i: (i, 0, 0)),
            pl.BlockSpec((1, 1, cout), lambda i: (0, 0, 0)),
            pl.BlockSpec((1, 1, cout), lambda i: (0, 0, 0)),
        ),
        out_specs=pl.BlockSpec((bpb2, s, cout), lambda i: (i, 0, 0)),
        out_shape=jax.ShapeDtypeStruct((n, s, cout), F32),
        compiler_params=cparams,
        cost_estimate=pl.CostEstimate(
            flops=3 * n * s * cout, transcendentals=0,
            bytes_accessed=6 * n * cout * s),
    )(conv2d, scale3, shift3)

    return jnp.transpose(out3.reshape(n, h, w, cout), (0, 3, 1, 2))


def kernel(x_nchw, w_oihw, b, gamma, beta):
    del b
    return _chess_conv_block(x_nchw, w_oihw, gamma, beta)
